# Initial kernel scaffold; baseline (speedup 1.0000x reference)
import jax
import jax.numpy as jnp
from jax.experimental import pallas as pl


def kernel(x, edge_index, batch, W1, b1, W2, b2, W3, b3, Wh, bh):
    raise NotImplementedError("write your pallas kernel here")


# speedup vs baseline: 9.3313x; 9.3313x over previous
import functools

import jax
import jax.numpy as jnp
from jax import lax
from jax.experimental import pallas as pl
from jax.experimental.pallas import tpu as pltpu
from jax.experimental.pallas import tpu_sc as plsc

N = 10000
E = 320000
D = 128
H = 128
G = 64

NC = 2
NS = 16
NW = NC * NS
K = 128

EPT = ((E + NW * K - 1) // (NW * K)) * K
E_PAD = EPT * NW
CH = EPT // K
RPT = (((N + 8 + NS - 1) // NS + K - 1) // K) * K
N_PAD = RPT * NS
RCH = RPT // K

NB = 25
BN = N // NB

_mesh = plsc.VectorSubcoreMesh(
    core_axis_name="c", subcore_axis_name="s", num_cores=NC, num_subcores=NS
)


def _deg_body(dst_hbm, out_hbm, dst_v, ones_v, zbuf_v, acc_sh, sem):
    c = lax.axis_index("c")
    s = lax.axis_index("s")
    wid = c * NS + s
    for j in range(K // 16):
        ones_v[pl.ds(j * 16, 16)] = jnp.ones((16,), jnp.float32)
    for j in range(RPT // 16):
        zbuf_v[pl.ds(j * 16, 16)] = jnp.zeros((16,), jnp.float32)
    pltpu.sync_copy(zbuf_v, acc_sh.at[pl.ds(s * RPT, RPT)])
    plsc.subcore_barrier()
    base = wid * EPT

    def body(i, carry):
        off = base + i * K
        pltpu.sync_copy(dst_hbm.at[pl.ds(off, K)], dst_v)
        pltpu.sync_copy(ones_v, acc_sh.at[dst_v], add=True)
        return carry

    lax.fori_loop(0, CH, body, 0)
    plsc.subcore_barrier()
    pltpu.sync_copy(acc_sh.at[pl.ds(s * RPT, RPT)], zbuf_v)
    pltpu.sync_copy(zbuf_v, out_hbm.at[pl.ds(c * N_PAD + s * RPT, RPT)])


_deg_call = pl.kernel(
    _deg_body,
    out_type=jax.ShapeDtypeStruct((NC * N_PAD,), jnp.float32),
    mesh=_mesh,
    scratch_types=[
        pltpu.VMEM((K,), jnp.int32),
        pltpu.VMEM((K,), jnp.float32),
        pltpu.VMEM((RPT,), jnp.float32),
        pltpu.VMEM_SHARED((N_PAD,), jnp.float32),
        pltpu.SemaphoreType.DMA,
    ],
)


def _agg_body(m_hbm, src_hbm, dst_hbm, out_hbm,
              src_v, dst_v, rows_v, acc_sh, sem):
    c = lax.axis_index("c")
    s = lax.axis_index("s")
    wid = c * NS + s

    def zrow(r, carry):
        for j in range(H // 16):
            rows_v[r, pl.ds(j * 16, 16)] = jnp.zeros((16,), jnp.float32)
        return carry

    lax.fori_loop(0, K, zrow, 0)
    for j in range(RCH):
        pltpu.sync_copy(rows_v, acc_sh.at[pl.ds(s * RPT + j * K, K)])
    plsc.subcore_barrier()
    base = wid * EPT

    def body(i, carry):
        off = base + i * K
        pltpu.sync_copy(src_hbm.at[pl.ds(off, K)], src_v)
        pltpu.sync_copy(dst_hbm.at[pl.ds(off, K)], dst_v)
        pltpu.async_copy(m_hbm.at[src_v], rows_v, sem).wait()
        pltpu.sync_copy(rows_v, acc_sh.at[dst_v], add=True)
        return carry

    lax.fori_loop(0, CH, body, 0)
    plsc.subcore_barrier()
    for j in range(RCH):
        pltpu.sync_copy(acc_sh.at[pl.ds(s * RPT + j * K, K)], rows_v)
        pltpu.sync_copy(rows_v, out_hbm.at[c, pl.ds(s * RPT + j * K, K)])


_agg_call = pl.kernel(
    _agg_body,
    out_type=jax.ShapeDtypeStruct((NC, N_PAD, H), jnp.float32),
    mesh=_mesh,
    scratch_types=[
        pltpu.VMEM((K,), jnp.int32),
        pltpu.VMEM((K,), jnp.int32),
        pltpu.VMEM((K, H), jnp.float32),
        pltpu.VMEM_SHARED((N_PAD, H), jnp.float32),
        pltpu.SemaphoreType.DMA,
    ],
)


def _k1_body(deg_ref, x_ref, w_ref, m_ref, dinv_ref):
    d = deg_ref[0] + deg_ref[1] + 1.0
    dv = lax.rsqrt(d)
    m = jnp.dot(x_ref[...], w_ref[...], preferred_element_type=jnp.float32)
    m_ref[...] = dv * m
    dinv_ref[...] = dv


_k1_call = pl.pallas_call(
    _k1_body,
    grid=(NB,),
    in_specs=[
        pl.BlockSpec((NC, BN, 1), lambda i: (0, i, 0)),
        pl.BlockSpec((BN, D), lambda i: (i, 0)),
        pl.BlockSpec((D, H), lambda i: (0, 0)),
    ],
    out_specs=[
        pl.BlockSpec((BN, H), lambda i: (i, 0)),
        pl.BlockSpec((BN, 1), lambda i: (i, 0)),
    ],
    out_shape=[
        jax.ShapeDtypeStruct((N, H), jnp.float32),
        jax.ShapeDtypeStruct((N, 1), jnp.float32),
    ],
)


def _layer_body(p_ref, m_ref, dinv_ref, b_ref, w_ref, out_ref):
    dv = dinv_ref[...]
    agg = dv * (p_ref[0] + p_ref[1] + m_ref[...]) + b_ref[...]
    h = jnp.maximum(agg, 0.0)
    out_ref[...] = dv * jnp.dot(h, w_ref[...], preferred_element_type=jnp.float32)


_layer_call = pl.pallas_call(
    _layer_body,
    grid=(NB,),
    in_specs=[
        pl.BlockSpec((NC, BN, H), lambda i: (0, i, 0)),
        pl.BlockSpec((BN, H), lambda i: (i, 0)),
        pl.BlockSpec((BN, 1), lambda i: (i, 0)),
        pl.BlockSpec((1, H), lambda i: (0, 0)),
        pl.BlockSpec((H, H), lambda i: (0, 0)),
    ],
    out_specs=pl.BlockSpec((BN, H), lambda i: (i, 0)),
    out_shape=jax.ShapeDtypeStruct((N, H), jnp.float32),
)


def _final_body(p_ref, m_ref, dinv_ref, b_ref, batch_ref, wh_ref, bh_ref,
                s_ref, c_ref, out_ref):
    i = pl.program_id(0)
    agg = dinv_ref[...] * (p_ref[0] + p_ref[1] + m_ref[...]) + b_ref[...]
    h = jnp.maximum(agg, 0.0)
    ids = lax.broadcasted_iota(jnp.int32, (BN, G), 1)
    oh = (batch_ref[0] == ids).astype(jnp.float32)

    @pl.when(i == 0)
    def _():
        s_ref[...] = jnp.zeros_like(s_ref)
        c_ref[...] = jnp.zeros_like(c_ref)

    s_ref[...] += lax.dot_general(
        oh, h, dimension_numbers=(((0,), (0,)), ((), ())),
        preferred_element_type=jnp.float32)
    c_ref[...] += jnp.sum(oh, axis=0)[:, None]

    @pl.when(i == NB - 1)
    def _():
        pooled = s_ref[...] / jnp.maximum(c_ref[...], 1.0)
        out_ref[...] = (
            jnp.dot(pooled, wh_ref[...], preferred_element_type=jnp.float32)
            + bh_ref[...]
        )


_final_call = pl.pallas_call(
    _final_body,
    grid=(NB,),
    in_specs=[
        pl.BlockSpec((NC, BN, H), lambda i: (0, i, 0)),
        pl.BlockSpec((BN, H), lambda i: (i, 0)),
        pl.BlockSpec((BN, 1), lambda i: (i, 0)),
        pl.BlockSpec((1, H), lambda i: (0, 0)),
        pl.BlockSpec((1, BN, 1), lambda i: (i, 0, 0)),
        pl.BlockSpec((H, 2), lambda i: (0, 0)),
        pl.BlockSpec((1, 2), lambda i: (0, 0)),
    ],
    out_specs=[
        pl.BlockSpec((G, H), lambda i: (0, 0)),
        pl.BlockSpec((G, 1), lambda i: (0, 0)),
        pl.BlockSpec((G, 2), lambda i: (0, 0)),
    ],
    out_shape=[
        jax.ShapeDtypeStruct((G, H), jnp.float32),
        jax.ShapeDtypeStruct((G, 1), jnp.float32),
        jax.ShapeDtypeStruct((G, 2), jnp.float32),
    ],
)


def kernel(x, edge_index, batch, W1, b1, W2, b2, W3, b3, Wh, bh):
    pad = E_PAD - E
    src = jnp.concatenate(
        [edge_index[0].astype(jnp.int32), jnp.zeros((pad,), jnp.int32)])
    dst = jnp.concatenate(
        [edge_index[1].astype(jnp.int32), jnp.full((pad,), N, jnp.int32)])

    deg = _deg_call(dst).reshape(NC, N_PAD)
    deg3 = deg[:, :N].reshape(NC, N, 1)

    m1, dinv = _k1_call(deg3, x, W1)
    p1 = _agg_call(m1, src, dst)
    m2 = _layer_call(p1[:, :N], m1, dinv, b1.reshape(1, H), W2)
    p2 = _agg_call(m2, src, dst)
    m3 = _layer_call(p2[:, :N], m2, dinv, b2.reshape(1, H), W3)
    p3 = _agg_call(m3, src, dst)

    batch3 = batch.astype(jnp.int32).reshape(NB, BN, 1)
    _, _, out = _final_call(p3[:, :N], m3, dinv, b3.reshape(1, H), batch3,
                            Wh, bh.reshape(1, 2))
    return out
